# baseline (speedup 1.0000x reference)
import jax
import jax.numpy as jnp
from jax.experimental import pallas as pl
from jax.experimental.pallas import tpu as pltpu

U = 10000
I = 10000
G = 1000
D = 128

MB = 200
NM = G // MB
RB = 3000
NR = (U + I + G) // RB


def _msg_kernel(uh_ref, ih_ref, ue_ref, ie_ref, wt_ref, b_ref, msg_ref):
    u_msg = jnp.dot(uh_ref[...], ue_ref[...],
                    preferred_element_type=jnp.float32)
    i_msg = jnp.dot(ih_ref[...], ie_ref[...],
                    preferred_element_type=jnp.float32)
    msg_ref[...] = (
        jnp.dot(u_msg, wt_ref[:D, :], preferred_element_type=jnp.float32)
        + jnp.dot(i_msg, wt_ref[D:, :], preferred_element_type=jnp.float32)
        + b_ref[...]
    )


def _norm_kernel(fh_ref, msg_ref, out_ref):
    out_ref[...] = jnp.dot(fh_ref[...], msg_ref[...],
                           preferred_element_type=jnp.float32)


def kernel(user_emb, item_emb, group_emb, user_hyper_graph,
           item_hyper_graph, full_hyper, W_agg, b_agg):
    wt = W_agg.T
    b2 = b_agg.reshape(1, D)

    msg = pl.pallas_call(
        _msg_kernel,
        grid=(NM,),
        in_specs=[
            pl.BlockSpec((MB, U), lambda m: (m, 0)),
            pl.BlockSpec((MB, I), lambda m: (m, 0)),
            pl.BlockSpec((U, D), lambda m: (0, 0)),
            pl.BlockSpec((I, D), lambda m: (0, 0)),
            pl.BlockSpec((2 * D, D), lambda m: (0, 0)),
            pl.BlockSpec((1, D), lambda m: (0, 0)),
        ],
        out_specs=pl.BlockSpec((MB, D), lambda m: (m, 0)),
        out_shape=jax.ShapeDtypeStruct((G, D), jnp.float32),
        compiler_params=pltpu.CompilerParams(
            dimension_semantics=("arbitrary",)),
    )(user_hyper_graph, item_hyper_graph, user_emb, item_emb, wt, b2)

    norm_emb = pl.pallas_call(
        _norm_kernel,
        grid=(NR,),
        in_specs=[
            pl.BlockSpec((RB, G), lambda r: (r, 0)),
            pl.BlockSpec((G, D), lambda r: (0, 0)),
        ],
        out_specs=pl.BlockSpec((RB, D), lambda r: (r, 0)),
        out_shape=jax.ShapeDtypeStruct((U + I + G, D), jnp.float32),
        compiler_params=pltpu.CompilerParams(
            dimension_semantics=("arbitrary",)),
    )(full_hyper, msg)

    return (norm_emb, msg)
